# baseline (speedup 1.0000x reference)
import jax
import jax.numpy as jnp
from jax import lax
from jax.experimental import pallas as pl
from jax.experimental.pallas import tpu as pltpu
from jax.experimental.pallas import tpu_sc as plsc

E = 16
TOPK = 2
D = 1024
F = 4096
T = 256

TR = 32
CAP = T * TOPK + E * TR
FT = 2048
NF = F // FT

NC = 2
NS = 16
NW = NC * NS

_i32 = jnp.int32
_f32 = jnp.float32


def _top2_body(l_ref, e0_ref, e1_ref, w0_ref, w1_ref):
    l = l_ref[...]
    iota = lax.broadcasted_iota(_i32, (T, E), 1)
    m0 = jnp.max(l, axis=1, keepdims=True)
    e0 = jnp.min(jnp.where(l == m0, iota, E), axis=1, keepdims=True)
    masked = jnp.where(iota == e0, -jnp.inf, l)
    m1 = jnp.max(masked, axis=1, keepdims=True)
    e1 = jnp.min(jnp.where(masked == m1, iota, E), axis=1, keepdims=True)
    w0 = 1.0 / (1.0 + jnp.exp(m1 - m0))
    e0_ref[...] = e0
    e1_ref[...] = e1
    w0_ref[...] = w0
    w1_ref[...] = 1.0 - w0


def _top2(logits):
    return pl.pallas_call(
        _top2_body,
        out_shape=(jax.ShapeDtypeStruct((T, 1), _i32),
                   jax.ShapeDtypeStruct((T, 1), _i32),
                   jax.ShapeDtypeStruct((T, 1), _f32),
                   jax.ShapeDtypeStruct((T, 1), _f32)),
    )(logits)


def _route_meta(e0, e1, w0, w1):
    smesh = plsc.ScalarSubcoreMesh(axis_name="score", num_cores=1)

    @pl.kernel(
        out_type=(jax.ShapeDtypeStruct((CAP,), _i32),
                  jax.ShapeDtypeStruct((CAP,), _f32),
                  jax.ShapeDtypeStruct((E,), _i32),
                  jax.ShapeDtypeStruct((E,), _i32),
                  jax.ShapeDtypeStruct((T,), _i32),
                  jax.ShapeDtypeStruct((T,), _i32)),
        mesh=smesh,
        scratch_types=[pltpu.SMEM((T,), _i32), pltpu.SMEM((T,), _i32),
                       pltpu.SMEM((T,), _f32), pltpu.SMEM((T,), _f32),
                       pltpu.SMEM((CAP,), _i32), pltpu.SMEM((CAP,), _f32),
                       pltpu.SMEM((E,), _i32), pltpu.SMEM((E,), _i32),
                       pltpu.SMEM((E,), _i32),
                       pltpu.SMEM((T,), _i32), pltpu.SMEM((T,), _i32),
                       pltpu.SemaphoreType.DMA],
    )
    def k2(e0_h, e1_h, w0_h, w1_h,
           tok_h, wts_h, as_h, nt_h, s0_h, s1_h,
           e0_s, e1_s, w0_s, w1_s, tok_s, wts_s,
           as_s, nt_s, fill_s, s0_s, s1_s, sem):
        pltpu.async_copy(e0_h, e0_s, sem).wait()
        pltpu.async_copy(e1_h, e1_s, sem).wait()
        pltpu.async_copy(w0_h, w0_s, sem).wait()
        pltpu.async_copy(w1_h, w1_s, sem).wait()

        @pl.loop(0, E)
        def _(e):
            fill_s[e] = 0

        @pl.loop(0, T)
        def _(t):
            a = e0_s[t]
            fill_s[a] += 1
            c = e1_s[t]
            fill_s[c] += 1

        run = 0
        for e in range(E):
            as_s[e] = run
            cnt = fill_s[e]
            ntl = (cnt + (TR - 1)) // TR
            nt_s[e] = ntl
            fill_s[e] = run
            run = run + ntl * TR

        @pl.loop(0, T)
        def _(t):
            e = e0_s[t]
            p = fill_s[e]
            tok_s[p] = t
            wts_s[p] = w0_s[t]
            s0_s[t] = p
            fill_s[e] = p + 1

        @pl.loop(0, T)
        def _(t):
            e = e1_s[t]
            p = fill_s[e]
            tok_s[p] = t
            wts_s[p] = w1_s[t]
            s1_s[t] = p
            fill_s[e] = p + 1

        pltpu.async_copy(tok_s, tok_h, sem).wait()
        pltpu.async_copy(wts_s, wts_h, sem).wait()
        pltpu.async_copy(as_s, as_h, sem).wait()
        pltpu.async_copy(nt_s, nt_h, sem).wait()
        pltpu.async_copy(s0_s, s0_h, sem).wait()
        pltpu.async_copy(s1_s, s1_h, sem).wait()

    return k2(e0, e1, w0, w1)


def _sc_gather(x, tok_ids):
    vmesh = plsc.VectorSubcoreMesh(core_axis_name="c", subcore_axis_name="s")
    bpw = CAP // NW

    @pl.kernel(
        out_type=jax.ShapeDtypeStruct((CAP, D), _f32),
        mesh=vmesh,
        scratch_types=[pltpu.VMEM((bpw,), _i32),
                       pltpu.VMEM((bpw, D), _f32),
                       pltpu.SemaphoreType.DMA],
    )
    def k3(x_hbm, idx_hbm, out_hbm, idx_v, rows_v, sem):
        wid = lax.axis_index("s") * NC + lax.axis_index("c")
        base = wid * bpw
        pltpu.sync_copy(idx_hbm.at[pl.ds(base, bpw)], idx_v)

        @pl.loop(0, bpw, step=16)
        def _(c):
            slc = (pl.ds(c, 16),)
            idx_v.at[*slc][...] = idx_v.at[*slc][...] & (T - 1)

        pltpu.async_copy(x_hbm.at[idx_v], rows_v, sem).wait()
        pltpu.sync_copy(rows_v, out_hbm.at[pl.ds(base, bpw)])

    return k3(x, tok_ids)


def _sc_combine(ys, slot0, slot1):
    vmesh = plsc.VectorSubcoreMesh(core_axis_name="c", subcore_axis_name="s")
    bpw = T // NW

    @pl.kernel(
        out_type=jax.ShapeDtypeStruct((T, D), _f32),
        mesh=vmesh,
        scratch_types=[pltpu.VMEM((bpw,), _i32),
                       pltpu.VMEM((bpw,), _i32),
                       pltpu.VMEM((bpw, D), _f32),
                       pltpu.VMEM((bpw, D), _f32),
                       pltpu.SemaphoreType.DMA],
    )
    def k5(ys_hbm, s0_hbm, s1_hbm, out_hbm, s0_v, s1_v, a_v, b_v, sem):
        wid = lax.axis_index("s") * NC + lax.axis_index("c")
        base = wid * bpw
        pltpu.sync_copy(s0_hbm.at[pl.ds(base, bpw)], s0_v)
        pltpu.sync_copy(s1_hbm.at[pl.ds(base, bpw)], s1_v)
        pltpu.async_copy(ys_hbm.at[s0_v], a_v, sem).wait()
        pltpu.async_copy(ys_hbm.at[s1_v], b_v, sem).wait()

        @pl.loop(0, bpw)
        def _(r):
            @pl.loop(0, D, step=16)
            def _(c):
                slc = (pl.ds(r, 1), pl.ds(c, 16))
                a_v.at[*slc][...] = a_v.at[*slc][...] + b_v.at[*slc][...]

        pltpu.sync_copy(a_v, out_hbm.at[pl.ds(base, bpw)])

    return k5(ys, slot0, slot1)


def _ffn_body(astart_ref, nt_ref, xs_ref, wts_ref, w1_ref, w3_ref, w2_ref,
              ys_ref):
    e = pl.program_id(0)
    f = pl.program_id(1)
    base = astart_ref[e]
    nt = nt_ref[e]

    def tile(i, carry):
        r0 = pl.multiple_of(base + i * TR, TR)
        xt = xs_ref[pl.ds(r0, TR), :]
        h = jnp.dot(xt, w1_ref[0], preferred_element_type=_f32)
        g = jnp.dot(xt, w3_ref[0], preferred_element_type=_f32)
        act = (h * jax.nn.sigmoid(h)) * g
        y = jnp.dot(act, w2_ref[0], preferred_element_type=_f32)
        y = y * wts_ref[pl.ds(r0, TR), :]

        @pl.when(f == 0)
        def _():
            ys_ref[pl.ds(r0, TR), :] = y

        @pl.when(f != 0)
        def _():
            ys_ref[pl.ds(r0, TR), :] = ys_ref[pl.ds(r0, TR), :] + y

        return carry

    jax.lax.fori_loop(0, nt, tile, 0)


def _expert_ffn(xs, wts, w1, w3, w2, astart, ntiles):
    return pl.pallas_call(
        _ffn_body,
        grid=(E, NF),
        in_specs=[
            pl.BlockSpec(memory_space=pltpu.SMEM),
            pl.BlockSpec(memory_space=pltpu.SMEM),
            pl.BlockSpec((CAP, D), lambda e, f: (0, 0)),
            pl.BlockSpec((CAP, 1), lambda e, f: (0, 0)),
            pl.BlockSpec((1, D, FT), lambda e, f: (e, 0, f)),
            pl.BlockSpec((1, D, FT), lambda e, f: (e, 0, f)),
            pl.BlockSpec((1, FT, D), lambda e, f: (e, f, 0)),
        ],
        out_specs=pl.BlockSpec((CAP, D), lambda e, f: (0, 0)),
        out_shape=jax.ShapeDtypeStruct((CAP, D), _f32),
    )(astart, ntiles, xs, wts, w1, w3, w2)


def kernel(hidden_states, gate_w, w1, w3, w2):
    b, s, d = hidden_states.shape
    x = hidden_states.reshape(-1, d)

    logits = x @ gate_w

    e0, e1, wn0, wn1 = _top2(logits)
    tok_ids, wts, astart, ntiles, slot0, slot1 = _route_meta(
        e0.reshape(T), e1.reshape(T), wn0.reshape(T), wn1.reshape(T))
    xs = _sc_gather(x, tok_ids)
    ys = _expert_ffn(xs, wts.reshape(CAP, 1), w1, w3, w2, astart, ntiles)
    out = _sc_combine(ys, slot0, slot1)
    return out.reshape(b, s, d).astype(hidden_states.dtype), logits


# speedup vs baseline: 1.1864x; 1.1864x over previous
import jax
import jax.numpy as jnp
from jax import lax
from jax.experimental import pallas as pl
from jax.experimental.pallas import tpu as pltpu
from jax.experimental.pallas import tpu_sc as plsc

E = 16
TOPK = 2
D = 1024
F = 4096
T = 256
A = T * TOPK

TR = 32
CAP = A + E * TR
FT = 1024
NF = F // FT

_i32 = jnp.int32
_f32 = jnp.float32
_HI = jax.lax.Precision.HIGHEST


def _top2_body(l_ref, e0_ref, e1_ref, w0_ref, w1_ref):
    l = l_ref[...]
    iota = lax.broadcasted_iota(_i32, (T, E), 1)
    m0 = jnp.max(l, axis=1, keepdims=True)
    e0 = jnp.min(jnp.where(l == m0, iota, E), axis=1, keepdims=True)
    masked = jnp.where(iota == e0, -jnp.inf, l)
    m1 = jnp.max(masked, axis=1, keepdims=True)
    e1 = jnp.min(jnp.where(masked == m1, iota, E), axis=1, keepdims=True)
    w0 = 1.0 / (1.0 + jnp.exp(m1 - m0))
    e0_ref[...] = e0
    e1_ref[...] = e1
    w0_ref[...] = w0
    w1_ref[...] = 1.0 - w0


def _top2(logits):
    return pl.pallas_call(
        _top2_body,
        out_shape=(jax.ShapeDtypeStruct((T, 1), _i32),
                   jax.ShapeDtypeStruct((T, 1), _i32),
                   jax.ShapeDtypeStruct((T, 1), _f32),
                   jax.ShapeDtypeStruct((T, 1), _f32)),
    )(logits)


def _histogram(e0, e1):
    smesh = plsc.ScalarSubcoreMesh(axis_name="score", num_cores=1)

    @pl.kernel(
        out_type=(jax.ShapeDtypeStruct((E,), _i32),
                  jax.ShapeDtypeStruct((E,), _i32)),
        mesh=smesh,
        scratch_types=[pltpu.SMEM((T,), _i32), pltpu.SMEM((T,), _i32),
                       pltpu.SMEM((E,), _i32), pltpu.SMEM((E,), _i32),
                       pltpu.SMEM((E,), _i32),
                       pltpu.SemaphoreType.DMA],
    )
    def k2(e0_h, e1_h, as_h, nt_h, e0_s, e1_s, as_s, nt_s, cnt_s, sem):
        pltpu.async_copy(e0_h, e0_s, sem).wait()
        pltpu.async_copy(e1_h, e1_s, sem).wait()

        @pl.loop(0, E)
        def _(e):
            cnt_s[e] = 0

        @pl.loop(0, T)
        def _(t):
            a = e0_s[t]
            cnt_s[a] += 1
            c = e1_s[t]
            cnt_s[c] += 1

        run = 0
        for e in range(E):
            as_s[e] = run
            ntl = (cnt_s[e] + (TR - 1)) // TR
            nt_s[e] = ntl
            run = run + ntl * TR

        pltpu.async_copy(as_s, as_h, sem).wait()
        pltpu.async_copy(nt_s, nt_h, sem).wait()

    return k2(e0, e1)


def _col_to_row(col):
    n = col.shape[0]
    eye = (lax.broadcasted_iota(_i32, (n, n), 0) ==
           lax.broadcasted_iota(_i32, (n, n), 1)).astype(_f32)
    return lax.dot_general(col, eye, (((0,), (0,)), ((), ())),
                           precision=_HI, preferred_element_type=_f32)


def _row_to_col(row):
    n = row.shape[1]
    eye = (lax.broadcasted_iota(_i32, (n, n), 0) ==
           lax.broadcasted_iota(_i32, (n, n), 1)).astype(_f32)
    return lax.dot_general(eye, row, (((1,), (1,)), ((), ())),
                           precision=_HI, preferred_element_type=_f32)


def _ffn_body(astart_ref, nt_ref, x_ref, e0_ref, e1_ref, w0_ref, w1_ref,
              asv_ref, w1w_ref, w3w_ref, w2w_ref, out_ref,
              xs_ref, ys_ref, wts_ref, u_ref):
    e = pl.program_id(0)
    f = pl.program_id(1)

    @pl.when(jnp.logical_and(e == 0, f == 0))
    def _prologue():
        eid = jnp.concatenate([e0_ref[...], e1_ref[...]], axis=0)
        wn = jnp.concatenate([w0_ref[...], w1_ref[...]], axis=0)
        eid_f = eid.astype(_f32)
        eid_row = _col_to_row(eid_f)

        sub_a = lax.broadcasted_iota(_i32, (A, A), 0)
        lane_a = lax.broadcasted_iota(_i32, (A, A), 1)
        same = (eid_f == eid_row) & (sub_a < lane_a)
        rank_row = lax.dot_general(
            jnp.ones((A, 1), _f32), same.astype(_f32),
            (((0,), (0,)), ((), ())), preferred_element_type=_f32)

        oh_t = (lax.broadcasted_iota(_i32, (E, A), 0).astype(_f32)
                == eid_row)
        astart_f = asv_ref[...].astype(_f32)
        astart_row = lax.dot_general(
            astart_f, oh_t.astype(_f32), (((0,), (0,)), ((), ())),
            precision=_HI, preferred_element_type=_f32)
        pos_row = astart_row + rank_row

        sel = (lax.broadcasted_iota(_i32, (CAP, A), 0).astype(_f32)
               == pos_row).astype(_f32)
        s_b = lax.broadcasted_iota(_i32, (A, T), 0)
        l_b = lax.broadcasted_iota(_i32, (A, T), 1)
        tokmap = ((s_b == l_b) | (s_b == l_b + T)).astype(_f32)
        gmat = jnp.dot(sel, tokmap, preferred_element_type=_f32)
        xs_ref[...] = jnp.dot(gmat, x_ref[...],
                              preferred_element_type=_f32)
        wts_ref[...] = jnp.dot(sel, wn, precision=_HI,
                               preferred_element_type=_f32)

        slot0 = _row_to_col(pos_row[:, :T])
        slot1 = _row_to_col(pos_row[:, T:])
        lane_p = lax.broadcasted_iota(_i32, (T, CAP), 1).astype(_f32)
        u_ref[...] = (jnp.where(lane_p == slot0, w0_ref[...], 0.0) +
                      jnp.where(lane_p == slot1, w1_ref[...], 0.0))
        ys_ref[...] = jnp.zeros((CAP, D), _f32)

    base = astart_ref[e]
    nt = nt_ref[e]

    def tile(i, carry):
        r0 = pl.multiple_of(base + i * TR, TR)
        xt = xs_ref[pl.ds(r0, TR), :]
        h = jnp.dot(xt, w1w_ref[0], preferred_element_type=_f32)
        g = jnp.dot(xt, w3w_ref[0], preferred_element_type=_f32)
        act = (h * jax.nn.sigmoid(h)) * g
        y = jnp.dot(act, w2w_ref[0], preferred_element_type=_f32)
        y = y * wts_ref[pl.ds(r0, TR), :]

        @pl.when(f == 0)
        def _():
            ys_ref[pl.ds(r0, TR), :] = y

        @pl.when(f != 0)
        def _():
            ys_ref[pl.ds(r0, TR), :] = ys_ref[pl.ds(r0, TR), :] + y

        return carry

    jax.lax.fori_loop(0, nt, tile, 0)

    @pl.when(jnp.logical_and(e == E - 1, f == NF - 1))
    def _epilogue():
        out_ref[...] = jnp.dot(u_ref[...], ys_ref[...],
                               preferred_element_type=_f32)


def _moe(x, e0, e1, w0, w1, astart_v, w1w, w3w, w2w, astart, ntiles):
    return pl.pallas_call(
        _ffn_body,
        grid=(E, NF),
        in_specs=[
            pl.BlockSpec(memory_space=pltpu.SMEM),
            pl.BlockSpec(memory_space=pltpu.SMEM),
            pl.BlockSpec((T, D), lambda e, f: (0, 0)),
            pl.BlockSpec((T, 1), lambda e, f: (0, 0)),
            pl.BlockSpec((T, 1), lambda e, f: (0, 0)),
            pl.BlockSpec((T, 1), lambda e, f: (0, 0)),
            pl.BlockSpec((T, 1), lambda e, f: (0, 0)),
            pl.BlockSpec((E, 1), lambda e, f: (0, 0)),
            pl.BlockSpec((1, D, FT), lambda e, f: (e, 0, f)),
            pl.BlockSpec((1, D, FT), lambda e, f: (e, 0, f)),
            pl.BlockSpec((1, FT, D), lambda e, f: (e, f, 0)),
        ],
        out_specs=pl.BlockSpec((T, D), lambda e, f: (0, 0)),
        out_shape=jax.ShapeDtypeStruct((T, D), _f32),
        scratch_shapes=[pltpu.VMEM((CAP, D), _f32),
                        pltpu.VMEM((CAP, D), _f32),
                        pltpu.VMEM((CAP, 1), _f32),
                        pltpu.VMEM((T, CAP), _f32)],
    )(astart, ntiles, x, e0, e1, w0, w1, astart_v, w1w, w3w, w2w)


def kernel(hidden_states, gate_w, w1, w3, w2):
    b, s, d = hidden_states.shape
    x = hidden_states.reshape(-1, d)

    logits = x @ gate_w

    e0, e1, wn0, wn1 = _top2(logits)
    astart, ntiles = _histogram(e0.reshape(T), e1.reshape(T))
    out = _moe(x, e0, e1, wn0, wn1, astart.reshape(E, 1),
               w1, w3, w2, astart, ntiles)
    return out.reshape(b, s, d).astype(hidden_states.dtype), logits


# speedup vs baseline: 1.1975x; 1.0093x over previous
import jax
import jax.numpy as jnp
from jax import lax
from jax.experimental import pallas as pl
from jax.experimental.pallas import tpu as pltpu
from jax.experimental.pallas import tpu_sc as plsc

E = 16
TOPK = 2
D = 1024
F = 4096
T = 256
A = T * TOPK

TR = 32
CAP = A + E * TR
FT = 1024
NF = F // FT

_i32 = jnp.int32
_f32 = jnp.float32
_HI = jax.lax.Precision.HIGHEST


def _top2_body(l_ref, e0_ref, e1_ref, w0_ref, w1_ref):
    l = l_ref[...]
    iota = lax.broadcasted_iota(_i32, (T, E), 1)
    m0 = jnp.max(l, axis=1, keepdims=True)
    e0 = jnp.min(jnp.where(l == m0, iota, E), axis=1, keepdims=True)
    masked = jnp.where(iota == e0, -jnp.inf, l)
    m1 = jnp.max(masked, axis=1, keepdims=True)
    e1 = jnp.min(jnp.where(masked == m1, iota, E), axis=1, keepdims=True)
    w0 = 1.0 / (1.0 + jnp.exp(m1 - m0))
    e0_ref[...] = e0
    e1_ref[...] = e1
    w0_ref[...] = w0
    w1_ref[...] = 1.0 - w0


def _top2(logits):
    return pl.pallas_call(
        _top2_body,
        out_shape=(jax.ShapeDtypeStruct((T, 1), _i32),
                   jax.ShapeDtypeStruct((T, 1), _i32),
                   jax.ShapeDtypeStruct((T, 1), _f32),
                   jax.ShapeDtypeStruct((T, 1), _f32)),
    )(logits)


def _histogram(e0, e1):
    smesh = plsc.ScalarSubcoreMesh(axis_name="score", num_cores=1)

    @pl.kernel(
        out_type=(jax.ShapeDtypeStruct((E,), _i32),
                  jax.ShapeDtypeStruct((E,), _i32)),
        mesh=smesh,
        scratch_types=[pltpu.SMEM((T,), _i32), pltpu.SMEM((T,), _i32),
                       pltpu.SMEM((E,), _i32), pltpu.SMEM((E,), _i32),
                       pltpu.SMEM((E,), _i32),
                       pltpu.SemaphoreType.DMA],
    )
    def k2(e0_h, e1_h, as_h, nt_h, e0_s, e1_s, as_s, nt_s, cnt_s, sem):
        pltpu.async_copy(e0_h, e0_s, sem).wait()
        pltpu.async_copy(e1_h, e1_s, sem).wait()

        @pl.loop(0, E)
        def _(e):
            cnt_s[e] = 0

        @pl.loop(0, T)
        def _(t):
            a = e0_s[t]
            cnt_s[a] += 1
            c = e1_s[t]
            cnt_s[c] += 1

        run = 0
        for e in range(E):
            as_s[e] = run
            ntl = (cnt_s[e] + (TR - 1)) // TR
            nt_s[e] = ntl
            run = run + ntl * TR

        pltpu.async_copy(as_s, as_h, sem).wait()
        pltpu.async_copy(nt_s, nt_h, sem).wait()

    return k2(e0, e1)


def _col_to_row(col):
    n = col.shape[0]
    eye = (lax.broadcasted_iota(_i32, (n, n), 0) ==
           lax.broadcasted_iota(_i32, (n, n), 1)).astype(_f32)
    return lax.dot_general(col, eye, (((0,), (0,)), ((), ())),
                           precision=_HI, preferred_element_type=_f32)


def _row_to_col(row):
    n = row.shape[1]
    eye = (lax.broadcasted_iota(_i32, (n, n), 0) ==
           lax.broadcasted_iota(_i32, (n, n), 1)).astype(_f32)
    return lax.dot_general(eye, row, (((1,), (1,)), ((), ())),
                           precision=_HI, preferred_element_type=_f32)


def _ffn_body(astart_ref, nt_ref, x_ref, e0_ref, e1_ref, w0_ref, w1_ref,
              w1w_ref, w3w_ref, w2w_ref, out_ref,
              xs_ref, ys_ref, u_ref):
    e = pl.program_id(0)
    f = pl.program_id(1)

    @pl.when(jnp.logical_and(e == 0, f == 0))
    def _prologue():
        eid = jnp.concatenate([e0_ref[...], e1_ref[...]], axis=0)
        eid_f = eid.astype(_f32)
        eid_row = _col_to_row(eid_f)
        ones_a = jnp.ones((A, 1), _f32)

        sub_a = lax.broadcasted_iota(_i32, (A, A), 0)
        lane_a = lax.broadcasted_iota(_i32, (A, A), 1)
        before = ((eid_f == eid_row) & (lane_a < sub_a)).astype(_f32)
        rank_col = lax.dot_general(before, ones_a, (((1,), (0,)), ((), ())),
                                   preferred_element_type=_f32)

        oh = (eid_f == lax.broadcasted_iota(_i32, (A, E), 1)
              .astype(_f32)).astype(_f32)
        counts_row = lax.dot_general(ones_a, oh, (((0,), (0,)), ((), ())),
                                     preferred_element_type=_f32)
        aligned_row = jnp.floor((counts_row + (TR - 1)) * (1.0 / TR)) * TR
        aligned_col = _row_to_col(aligned_row)
        m2 = (lax.broadcasted_iota(_i32, (A, E), 1).astype(_f32)
              < eid_f).astype(_f32)
        astart_col = lax.dot_general(m2, aligned_col,
                                     (((1,), (0,)), ((), ())),
                                     preferred_element_type=_f32)
        pos_col = astart_col + rank_col

        pa = (pos_col == lax.broadcasted_iota(_i32, (A, CAP), 1)
              .astype(_f32)).astype(_f32)
        s_b = lax.broadcasted_iota(_i32, (A, T), 0)
        l_b = lax.broadcasted_iota(_i32, (A, T), 1)
        tokmap = ((s_b == l_b) | (s_b == l_b + T)).astype(_f32)
        gmat = lax.dot_general(pa, tokmap, (((0,), (0,)), ((), ())),
                               preferred_element_type=_f32)
        xs_ref[...] = jnp.dot(gmat, x_ref[...],
                              preferred_element_type=_f32)

        slot0 = pos_col[:T, :]
        slot1 = pos_col[T:, :]
        lane_p = lax.broadcasted_iota(_i32, (T, CAP), 1).astype(_f32)
        u_ref[...] = (jnp.where(lane_p == slot0, w0_ref[...], 0.0) +
                      jnp.where(lane_p == slot1, w1_ref[...], 0.0))
        ys_ref[...] = jnp.zeros((CAP, D), _f32)

    base = astart_ref[e]
    nt = nt_ref[e]

    def tile(i, carry):
        r0 = pl.multiple_of(base + i * TR, TR)
        xt = xs_ref[pl.ds(r0, TR), :]
        h = jnp.dot(xt, w1w_ref[0], preferred_element_type=_f32)
        g = jnp.dot(xt, w3w_ref[0], preferred_element_type=_f32)
        act = (h * jax.nn.sigmoid(h)) * g
        y = jnp.dot(act, w2w_ref[0], preferred_element_type=_f32)

        @pl.when(f == 0)
        def _():
            ys_ref[pl.ds(r0, TR), :] = y

        @pl.when(f != 0)
        def _():
            ys_ref[pl.ds(r0, TR), :] = ys_ref[pl.ds(r0, TR), :] + y

        return carry

    jax.lax.fori_loop(0, nt, tile, 0)

    @pl.when(jnp.logical_and(e == E - 1, f == NF - 1))
    def _epilogue():
        out_ref[...] = jnp.dot(u_ref[...], ys_ref[...],
                               preferred_element_type=_f32)


def _moe(x, e0, e1, w0, w1, w1w, w3w, w2w, astart, ntiles):
    return pl.pallas_call(
        _ffn_body,
        grid=(E, NF),
        in_specs=[
            pl.BlockSpec(memory_space=pltpu.SMEM),
            pl.BlockSpec(memory_space=pltpu.SMEM),
            pl.BlockSpec((T, D), lambda e, f: (0, 0)),
            pl.BlockSpec((T, 1), lambda e, f: (0, 0)),
            pl.BlockSpec((T, 1), lambda e, f: (0, 0)),
            pl.BlockSpec((T, 1), lambda e, f: (0, 0)),
            pl.BlockSpec((T, 1), lambda e, f: (0, 0)),
            pl.BlockSpec((1, D, FT), lambda e, f: (e, 0, f)),
            pl.BlockSpec((1, D, FT), lambda e, f: (e, 0, f)),
            pl.BlockSpec((1, FT, D), lambda e, f: (e, f, 0)),
        ],
        out_specs=pl.BlockSpec((T, D), lambda e, f: (0, 0)),
        out_shape=jax.ShapeDtypeStruct((T, D), _f32),
        scratch_shapes=[pltpu.VMEM((CAP, D), _f32),
                        pltpu.VMEM((CAP, D), _f32),
                        pltpu.VMEM((T, CAP), _f32)],
    )(astart, ntiles, x, e0, e1, w0, w1, w1w, w3w, w2w)


def kernel(hidden_states, gate_w, w1, w3, w2):
    b, s, d = hidden_states.shape
    x = hidden_states.reshape(-1, d)

    logits = x @ gate_w

    e0, e1, wn0, wn1 = _top2(logits)
    astart, ntiles = _histogram(e0.reshape(T), e1.reshape(T))
    out = _moe(x, e0, e1, wn0, wn1, w1, w3, w2, astart, ntiles)
    return out.reshape(b, s, d).astype(hidden_states.dtype), logits


# speedup vs baseline: 1.2008x; 1.0027x over previous
import jax
import jax.numpy as jnp
from jax import lax
from jax.experimental import pallas as pl
from jax.experimental.pallas import tpu as pltpu
from jax.experimental.pallas import tpu_sc as plsc

E = 16
TOPK = 2
D = 1024
F = 4096
T = 256
A = T * TOPK

TR = 32
CAP = A + E * TR
FT = 1024
NF = F // FT

_i32 = jnp.int32
_f32 = jnp.float32
_HI = jax.lax.Precision.HIGHEST


def _top2_body(l_ref, e0_ref, e1_ref, w0_ref, w1_ref):
    l = l_ref[...]
    iota = lax.broadcasted_iota(_i32, (T, E), 1)
    m0 = jnp.max(l, axis=1, keepdims=True)
    e0 = jnp.min(jnp.where(l == m0, iota, E), axis=1, keepdims=True)
    masked = jnp.where(iota == e0, -jnp.inf, l)
    m1 = jnp.max(masked, axis=1, keepdims=True)
    e1 = jnp.min(jnp.where(masked == m1, iota, E), axis=1, keepdims=True)
    w0 = 1.0 / (1.0 + jnp.exp(m1 - m0))
    e0_ref[...] = e0
    e1_ref[...] = e1
    w0_ref[...] = w0
    w1_ref[...] = 1.0 - w0


def _top2(logits):
    return pl.pallas_call(
        _top2_body,
        out_shape=(jax.ShapeDtypeStruct((T, 1), _i32),
                   jax.ShapeDtypeStruct((T, 1), _i32),
                   jax.ShapeDtypeStruct((T, 1), _f32),
                   jax.ShapeDtypeStruct((T, 1), _f32)),
    )(logits)


def _histogram(e0, e1):
    smesh = plsc.ScalarSubcoreMesh(axis_name="score", num_cores=1)

    @pl.kernel(
        out_type=(jax.ShapeDtypeStruct((E,), _i32),
                  jax.ShapeDtypeStruct((E,), _i32)),
        mesh=smesh,
        scratch_types=[pltpu.SMEM((T,), _i32), pltpu.SMEM((T,), _i32),
                       pltpu.SMEM((E,), _i32), pltpu.SMEM((E,), _i32),
                       pltpu.SMEM((E,), _i32),
                       pltpu.SemaphoreType.DMA],
    )
    def k2(e0_h, e1_h, as_h, nt_h, e0_s, e1_s, as_s, nt_s, cnt_s, sem):
        pltpu.async_copy(e0_h, e0_s, sem).wait()
        pltpu.async_copy(e1_h, e1_s, sem).wait()

        @pl.loop(0, E)
        def _(e):
            cnt_s[e] = 0

        @pl.loop(0, T)
        def _(t):
            a = e0_s[t]
            cnt_s[a] += 1
            c = e1_s[t]
            cnt_s[c] += 1

        run = 0
        for e in range(E):
            as_s[e] = run
            ntl = (cnt_s[e] + (TR - 1)) // TR
            nt_s[e] = ntl
            run = run + ntl * TR

        pltpu.async_copy(as_s, as_h, sem).wait()
        pltpu.async_copy(nt_s, nt_h, sem).wait()

    return k2(e0, e1)


def _col_to_row(col):
    n = col.shape[0]
    eye = (lax.broadcasted_iota(_i32, (n, n), 0) ==
           lax.broadcasted_iota(_i32, (n, n), 1)).astype(_f32)
    return lax.dot_general(col, eye, (((0,), (0,)), ((), ())),
                           precision=_HI, preferred_element_type=_f32)


def _row_to_col(row):
    n = row.shape[1]
    eye = (lax.broadcasted_iota(_i32, (n, n), 0) ==
           lax.broadcasted_iota(_i32, (n, n), 1)).astype(_f32)
    return lax.dot_general(eye, row, (((1,), (1,)), ((), ())),
                           precision=_HI, preferred_element_type=_f32)


def _ffn_body(astart_ref, nt_ref, x_ref, e0_ref, e1_ref, w0_ref, w1_ref,
              w1w_ref, w3w_ref, w2w_ref, out_ref,
              xs_ref, ys_ref, u_ref):
    e = pl.program_id(0)
    f = pl.program_id(1)

    @pl.when(jnp.logical_and(e == 0, f == 0))
    def _prologue():
        eid = jnp.concatenate([e0_ref[...], e1_ref[...]], axis=0)
        eid_f = eid.astype(_f32)
        eid_row = _col_to_row(eid_f)
        ones_a = jnp.ones((A, 1), _f32)

        sub_a = lax.broadcasted_iota(_i32, (A, A), 0)
        lane_a = lax.broadcasted_iota(_i32, (A, A), 1)
        before = ((eid_f == eid_row) & (lane_a < sub_a)).astype(_f32)
        rank_col = lax.dot_general(before, ones_a, (((1,), (0,)), ((), ())),
                                   preferred_element_type=_f32)

        oh = (eid_f == lax.broadcasted_iota(_i32, (A, E), 1)
              .astype(_f32)).astype(_f32)
        counts_row = lax.dot_general(ones_a, oh, (((0,), (0,)), ((), ())),
                                     preferred_element_type=_f32)
        aligned_row = jnp.floor((counts_row + (TR - 1)) * (1.0 / TR)) * TR
        aligned_col = _row_to_col(aligned_row)
        m2 = (lax.broadcasted_iota(_i32, (A, E), 1).astype(_f32)
              < eid_f).astype(_f32)
        astart_col = lax.dot_general(m2, aligned_col,
                                     (((1,), (0,)), ((), ())),
                                     preferred_element_type=_f32)
        pos_col = astart_col + rank_col

        pa = (pos_col == lax.broadcasted_iota(_i32, (A, CAP), 1)
              .astype(_f32)).astype(_f32)
        s_b = lax.broadcasted_iota(_i32, (A, T), 0)
        l_b = lax.broadcasted_iota(_i32, (A, T), 1)
        tokmap = ((s_b == l_b) | (s_b == l_b + T)).astype(_f32)
        gmat = lax.dot_general(pa, tokmap, (((0,), (0,)), ((), ())),
                               preferred_element_type=_f32)
        xs_ref[...] = jnp.dot(gmat, x_ref[...],
                              preferred_element_type=_f32)

        slot0 = pos_col[:T, :]
        slot1 = pos_col[T:, :]
        lane_p = lax.broadcasted_iota(_i32, (T, CAP), 1).astype(_f32)
        u_ref[...] = (jnp.where(lane_p == slot0, w0_ref[...], 0.0) +
                      jnp.where(lane_p == slot1, w1_ref[...], 0.0))
        ys_ref[...] = jnp.zeros((CAP, D), _f32)

    base = astart_ref[e]
    nt = nt_ref[e]

    def tile(i, carry):
        r0 = pl.multiple_of(base + i * TR, TR)
        xt = xs_ref[pl.ds(r0, TR), :]
        h = jnp.dot(xt, w1w_ref[0], preferred_element_type=_f32)
        g = jnp.dot(xt, w3w_ref[0], preferred_element_type=_f32)
        act = (h * jax.nn.sigmoid(h)) * g
        y = jnp.dot(act, w2w_ref[0], preferred_element_type=_f32)
        ys_ref[pl.ds(r0, TR), :] = ys_ref[pl.ds(r0, TR), :] + y
        return carry

    jax.lax.fori_loop(0, nt, tile, 0)

    @pl.when(jnp.logical_and(e == E - 1, f == NF - 1))
    def _epilogue():
        out_ref[...] = jnp.dot(u_ref[...], ys_ref[...],
                               preferred_element_type=_f32)


def _moe(x, e0, e1, w0, w1, w1w, w3w, w2w, astart, ntiles):
    return pl.pallas_call(
        _ffn_body,
        grid=(E, NF),
        in_specs=[
            pl.BlockSpec(memory_space=pltpu.SMEM),
            pl.BlockSpec(memory_space=pltpu.SMEM),
            pl.BlockSpec((T, D), lambda e, f: (0, 0)),
            pl.BlockSpec((T, 1), lambda e, f: (0, 0)),
            pl.BlockSpec((T, 1), lambda e, f: (0, 0)),
            pl.BlockSpec((T, 1), lambda e, f: (0, 0)),
            pl.BlockSpec((T, 1), lambda e, f: (0, 0)),
            pl.BlockSpec((1, D, FT), lambda e, f: (e, 0, f)),
            pl.BlockSpec((1, D, FT), lambda e, f: (e, 0, f)),
            pl.BlockSpec((1, FT, D), lambda e, f: (e, f, 0)),
        ],
        out_specs=pl.BlockSpec((T, D), lambda e, f: (0, 0)),
        out_shape=jax.ShapeDtypeStruct((T, D), _f32),
        scratch_shapes=[pltpu.VMEM((CAP, D), _f32),
                        pltpu.VMEM((CAP, D), _f32),
                        pltpu.VMEM((T, CAP), _f32)],
    )(astart, ntiles, x, e0, e1, w0, w1, w1w, w3w, w2w)


def kernel(hidden_states, gate_w, w1, w3, w2):
    b, s, d = hidden_states.shape
    x = hidden_states.reshape(-1, d)

    logits = x @ gate_w

    e0, e1, wn0, wn1 = _top2(logits)
    astart, ntiles = _histogram(e0.reshape(T), e1.reshape(T))
    out = _moe(x, e0, e1, wn0, wn1, w1, w3, w2, astart, ntiles)
    return out.reshape(b, s, d).astype(hidden_states.dtype), logits
